# Initial kernel scaffold; baseline (speedup 1.0000x reference)
import jax
import jax.numpy as jnp
from jax.experimental import pallas as pl


def kernel(x_a, x_b, edge_index_ab, edge_index_ba, batch_a, batch_b, emb_a, emb_b, Wl_ab_0, bl_ab_0, Wr_ab_0, Wl_ba_0, bl_ba_0, Wr_ba_0, Wl_ab_1, bl_ab_1, Wr_ab_1, Wl_ba_1, bl_ba_1, Wr_ba_1, W1, b1, W2, b2):
    raise NotImplementedError("write your pallas kernel here")


# speedup vs baseline: 6.3514x; 6.3514x over previous
import functools

import jax
import jax.numpy as jnp
from jax import lax
from jax.experimental import pallas as pl
from jax.experimental.pallas import tpu as pltpu
from jax.experimental.pallas import tpu_sc as plsc

N = 10000
E = 320000
D = 128
G = 8

NC = 2
NS = 16
NP = 10240
ROWS_T = NP // NS
CH = 80
NCH = E // (NS * CH)
KB = 50
NBLK = NCH // KB
BN = 640
NB = NP // BN

_mesh = plsc.VectorSubcoreMesh(core_axis_name="c", subcore_axis_name="s")


def _embed_body(emb, xidx, out, x_v, rows_v, sem):
    cid = lax.axis_index("c")
    sid = lax.axis_index("s")
    pltpu.sync_copy(xidx.at[cid, sid], x_v)
    for q in range(5):
        pltpu.async_copy(emb.at[x_v.at[q]], rows_v, sem).wait()
        pltpu.sync_copy(
            rows_v,
            out.at[pl.ds(cid * NP + sid * ROWS_T + q * 128, 128)])


_embed = functools.partial(
    pl.kernel, _embed_body, mesh=_mesh,
    out_type=jax.ShapeDtypeStruct((2 * NP, D), jnp.float32),
    scratch_types=[
        pltpu.VMEM((5, 128), jnp.int32),
        pltpu.VMEM((128, D), jnp.float32),
        pltpu.SemaphoreType.DMA,
    ],
)()


def _counts_body(dst_idx, out, dst_v, ones_v, z_v, acc):
    cid = lax.axis_index("c")
    sid = lax.axis_index("s")
    for k in range(ROWS_T // 16):
        z_v[pl.ds(k * 16, 16)] = jnp.zeros((16,), jnp.float32)
    for k in range(CH // 16):
        ones_v[pl.ds(k * 16, 16)] = jnp.ones((16,), jnp.float32)
    pltpu.sync_copy(z_v, acc.at[pl.ds(sid * ROWS_T, ROWS_T)])
    plsc.subcore_barrier()

    def blk(bi, carry):
        pltpu.sync_copy(dst_idx.at[cid, sid, bi], dst_v)

        def step(j, carry2):
            pltpu.sync_copy(ones_v, acc.at[dst_v.at[j]], add=True)
            return carry2

        lax.fori_loop(0, KB, step, 0)
        return carry

    lax.fori_loop(0, NBLK, blk, 0)
    plsc.subcore_barrier()
    pltpu.sync_copy(acc.at[pl.ds(sid * ROWS_T, ROWS_T)],
                    out.at[pl.ds(cid * NP + sid * ROWS_T, ROWS_T)])


_counts = functools.partial(
    pl.kernel, _counts_body, mesh=_mesh,
    out_type=jax.ShapeDtypeStruct((2 * NP,), jnp.float32),
    scratch_types=[
        pltpu.VMEM((KB, CH), jnp.int32),
        pltpu.VMEM((CH,), jnp.float32),
        pltpu.VMEM((ROWS_T,), jnp.float32),
        pltpu.VMEM_SHARED((NP,), jnp.float32),
    ],
)()


def _segsum_body(tables, src_idx, dst_idx, out,
                 src_v, dst_v, rows_v, z_v, acc, sem):
    cid = lax.axis_index("c")
    sid = lax.axis_index("s")

    def zrow(i, carry):
        for k in range(D // 16):
            z_v[i, pl.ds(k * 16, 16)] = jnp.zeros((16,), jnp.float32)
        return carry

    lax.fori_loop(0, 64, zrow, 0)
    for q in range(ROWS_T // 64):
        pltpu.sync_copy(z_v, acc.at[pl.ds(sid * ROWS_T + q * 64, 64)])
    plsc.subcore_barrier()

    def blk(bi, carry):
        pltpu.sync_copy(src_idx.at[cid, sid, bi], src_v)
        pltpu.sync_copy(dst_idx.at[cid, sid, bi], dst_v)

        def step(j, carry2):
            pltpu.async_copy(tables.at[src_v.at[j]], rows_v, sem).wait()
            pltpu.sync_copy(rows_v, acc.at[dst_v.at[j]], add=True)
            return carry2

        lax.fori_loop(0, KB, step, 0)
        return carry

    lax.fori_loop(0, NBLK, blk, 0)
    plsc.subcore_barrier()
    pltpu.sync_copy(acc.at[pl.ds(sid * ROWS_T, ROWS_T)],
                    out.at[pl.ds(cid * NP + sid * ROWS_T, ROWS_T)])


_segsum = functools.partial(
    pl.kernel, _segsum_body, mesh=_mesh,
    out_type=jax.ShapeDtypeStruct((2 * NP, D), jnp.float32),
    scratch_types=[
        pltpu.VMEM((KB, CH), jnp.int32),
        pltpu.VMEM((KB, CH), jnp.int32),
        pltpu.VMEM((CH, D), jnp.float32),
        pltpu.VMEM((64, D), jnp.float32),
        pltpu.VMEM_SHARED((NP, D), jnp.float32),
        pltpu.SemaphoreType.DMA,
    ],
)()


def _dense_body(s_ref, cnt_ref, h_ref, wl_ref, bl_ref, wr_ref, o_ref):
    s = s_ref[0]
    c = cnt_ref[0, 0, 0]
    mean = s / jnp.maximum(c, 1.0)[:, None]
    o_ref[0] = (jnp.dot(mean, wl_ref[0], preferred_element_type=jnp.float32)
                + bl_ref[0, 0][None, :]
                + jnp.dot(h_ref[0], wr_ref[0],
                          preferred_element_type=jnp.float32))


def _dense(s, cnt4, h, wl, bl, wr):
    return pl.pallas_call(
        _dense_body,
        grid=(2, NB),
        in_specs=[
            pl.BlockSpec((1, BN, D), lambda t, b: (t, b, 0)),
            pl.BlockSpec((1, 1, 1, BN), lambda t, b: (t, b, 0, 0)),
            pl.BlockSpec((1, BN, D), lambda t, b: (t, b, 0)),
            pl.BlockSpec((1, D, D), lambda t, b: (t, 0, 0)),
            pl.BlockSpec((1, 1, D), lambda t, b: (t, 0, 0)),
            pl.BlockSpec((1, D, D), lambda t, b: (t, 0, 0)),
        ],
        out_specs=pl.BlockSpec((1, BN, D), lambda t, b: (t, b, 0)),
        out_shape=jax.ShapeDtypeStruct((2, NP, D), jnp.float32),
    )(s, cnt4, h, wl, bl, wr)


def _final_body(s_ref, cnt_ref, h_ref, wl_ref, bl_ref, wr_ref, batch_ref,
                w1_ref, b1_ref, w2_ref, b2_ref, o_ref, acc_ref):
    t = pl.program_id(0)
    b = pl.program_id(1)

    @pl.when(jnp.logical_and(t == 0, b == 0))
    def _():
        acc_ref[...] = jnp.zeros_like(acc_ref)

    s = s_ref[0]
    c = cnt_ref[0, 0, 0]
    mean = s / jnp.maximum(c, 1.0)[:, None]
    hn = (jnp.dot(mean, wl_ref[0], preferred_element_type=jnp.float32)
          + bl_ref[0, 0][None, :]
          + jnp.dot(h_ref[0], wr_ref[0], preferred_element_type=jnp.float32))
    ids = batch_ref[0, 0, 0]
    oh = (ids[None, :] ==
          lax.broadcasted_iota(jnp.int32, (G, BN), 0)).astype(jnp.float32)
    acc_ref[...] += jnp.dot(oh, hn, preferred_element_type=jnp.float32)

    @pl.when(jnp.logical_and(t == 1, b == NB - 1))
    def _():
        g = acc_ref[...]
        g = jnp.maximum(
            jnp.dot(g, w1_ref[...], preferred_element_type=jnp.float32)
            + b1_ref[0][None, :], 0.0)
        o_ref[...] = (jnp.dot(g, w2_ref[...],
                              preferred_element_type=jnp.float32)
                      + b2_ref[0][None, :])


def _final(s, cnt4, h, wl, bl, wr, batch4, w1, b1, w2, b2):
    return pl.pallas_call(
        _final_body,
        grid=(2, NB),
        in_specs=[
            pl.BlockSpec((1, BN, D), lambda t, b: (t, b, 0)),
            pl.BlockSpec((1, 1, 1, BN), lambda t, b: (t, b, 0, 0)),
            pl.BlockSpec((1, BN, D), lambda t, b: (t, b, 0)),
            pl.BlockSpec((1, D, D), lambda t, b: (t, 0, 0)),
            pl.BlockSpec((1, 1, D), lambda t, b: (t, 0, 0)),
            pl.BlockSpec((1, D, D), lambda t, b: (t, 0, 0)),
            pl.BlockSpec((1, 1, 1, BN), lambda t, b: (t, b, 0, 0)),
            pl.BlockSpec((D, D), lambda t, b: (0, 0)),
            pl.BlockSpec((1, D), lambda t, b: (0, 0)),
            pl.BlockSpec((D, D), lambda t, b: (0, 0)),
            pl.BlockSpec((1, D), lambda t, b: (0, 0)),
        ],
        out_specs=pl.BlockSpec((G, D), lambda t, b: (0, 0)),
        out_shape=jax.ShapeDtypeStruct((G, D), jnp.float32),
        scratch_shapes=[pltpu.VMEM((G, D), jnp.float32)],
    )(s, cnt4, h, wl, bl, wr, batch4, w1, b1, w2, b2)


def kernel(x_a, x_b, edge_index_ab, edge_index_ba, batch_a, batch_b,
           emb_a, emb_b,
           Wl_ab_0, bl_ab_0, Wr_ab_0, Wl_ba_0, bl_ba_0, Wr_ba_0,
           Wl_ab_1, bl_ab_1, Wr_ab_1, Wl_ba_1, bl_ba_1, Wr_ba_1,
           W1, b1, W2, b2):
    zpadD = jnp.zeros((NP - N, D), jnp.float32)
    embt = jnp.concatenate([emb_a, zpadD, emb_b, zpadD], axis=0)
    zpadI = jnp.zeros((NP - N,), jnp.int32)
    x_stack = jnp.stack([
        jnp.concatenate([x_a, zpadI]),
        jnp.concatenate([x_b + NP, zpadI + NP]),
    ]).reshape(2, NS, 5, 128)
    src_stack = jnp.stack([
        (edge_index_ba[0] + NP).reshape(NS, NBLK, KB, CH),
        edge_index_ab[0].reshape(NS, NBLK, KB, CH)])
    dst_stack = jnp.stack([
        edge_index_ba[1].reshape(NS, NBLK, KB, CH),
        edge_index_ab[1].reshape(NS, NBLK, KB, CH)])

    h0 = _embed(embt, x_stack)
    cnt = _counts(dst_stack)
    cnt4 = cnt.reshape(2, NB, 1, BN)

    wl0 = jnp.stack([Wl_ba_0, Wl_ab_0])
    bl0 = jnp.stack([bl_ba_0, bl_ab_0]).reshape(2, 1, D)
    wr0 = jnp.stack([Wr_ba_0, Wr_ab_0])
    wl1 = jnp.stack([Wl_ba_1, Wl_ab_1])
    bl1 = jnp.stack([bl_ba_1, bl_ab_1]).reshape(2, 1, D)
    wr1 = jnp.stack([Wr_ba_1, Wr_ab_1])
    bpad = jnp.full((NP - N,), G, jnp.int32)
    batch4 = jnp.stack([
        jnp.concatenate([batch_a, bpad]),
        jnp.concatenate([batch_b, bpad]),
    ]).reshape(2, NB, 1, BN)

    s0 = _segsum(h0, src_stack, dst_stack)
    h1 = _dense(s0.reshape(2, NP, D), cnt4, h0.reshape(2, NP, D),
                wl0, bl0, wr0).reshape(2 * NP, D)
    s1 = _segsum(h1, src_stack, dst_stack)
    g = _final(s1.reshape(2, NP, D), cnt4, h1.reshape(2, NP, D),
               wl1, bl1, wr1, batch4,
               W1, b1.reshape(1, D), W2, b2.reshape(1, D))
    return g
